# Initial kernel scaffold; baseline (speedup 1.0000x reference)
import jax
import jax.numpy as jnp
from jax.experimental import pallas as pl


def kernel(x, edge_2body, edge_3body, edge_2bodySelf, edge_1body, edge_attr_2body, edge_attr_3body, edge_attr_2bodySelf, edge_attr_1body, W1_2b, b1_2b, W2_2b, b2_2b, W1_3b, b1_3b, W2_3b, b2_3b, W1_s, b1_s, W2_s, b2_s):
    raise NotImplementedError("write your pallas kernel here")


# speedup vs baseline: 1.1138x; 1.1138x over previous
import functools

import jax
import jax.numpy as jnp
from jax.experimental import pallas as pl


def _edge_mlp_kernel(d_ref, a_ref, W1_ref, b1_ref, W2_ref, b2_ref, y_ref):
    d = d_ref[...]
    a = a_ref[...]
    h = jnp.maximum(jnp.dot(d, W1_ref[...], preferred_element_type=jnp.float32)
                    + b1_ref[...][None, :], 0.0)
    o = jnp.dot(h, W2_ref[...], preferred_element_type=jnp.float32) + b2_ref[...][None, :]
    cols = []
    for i in range(3):
        cols.append(jnp.sum(o[:, 3 * i:3 * i + 3] * a, axis=1, keepdims=True))
    y_ref[...] = jnp.concatenate(cols, axis=1)


def _edge_mlp(d, a, W1, b1, W2, b2, blk):
    e, din = d.shape
    assert e % blk == 0, (e, blk)
    grid = e // blk
    return pl.pallas_call(
        _edge_mlp_kernel,
        grid=(grid,),
        in_specs=[
            pl.BlockSpec((blk, din), lambda i: (i, 0)),
            pl.BlockSpec((blk, 3), lambda i: (i, 0)),
            pl.BlockSpec(W1.shape, lambda i: (0, 0)),
            pl.BlockSpec(b1.shape, lambda i: (0,)),
            pl.BlockSpec(W2.shape, lambda i: (0, 0)),
            pl.BlockSpec(b2.shape, lambda i: (0,)),
        ],
        out_specs=pl.BlockSpec((blk, 3), lambda i: (i, 0)),
        out_shape=jax.ShapeDtypeStruct((e, 3), jnp.float32),
    )(d, a, W1, b1, W2, b2)


def kernel(x, edge_2body, edge_3body, edge_2bodySelf, edge_1body,
           edge_attr_2body, edge_attr_3body, edge_attr_2bodySelf, edge_attr_1body,
           W1_2b, b1_2b, W2_2b, b2_2b,
           W1_3b, b1_3b, W2_3b, b2_3b,
           W1_s, b1_s, W2_s, b2_s):
    n = x.shape[0]
    d2 = jnp.take(x, edge_2body[0], axis=0) - jnp.take(x, edge_2body[1], axis=0)
    y2 = _edge_mlp(d2, edge_attr_2body, W1_2b, b1_2b, W2_2b, b2_2b, blk=4000)
    v2 = jax.ops.segment_sum(y2, edge_2body[1], num_segments=n)
    xj = jnp.take(x, edge_3body[0], axis=0)
    xk = jnp.take(x, edge_3body[1], axis=0)
    xi = jnp.take(x, edge_3body[2], axis=0)
    d3 = jnp.concatenate((xk - xj, xi - xk), axis=1)
    y3 = _edge_mlp(d3, edge_attr_3body, W1_3b, b1_3b, W2_3b, b2_3b, blk=4000)
    v3 = jax.ops.segment_sum(y3, edge_3body[2], num_segments=n)
    ds = jnp.take(x, edge_2bodySelf[0], axis=0) - jnp.take(x, edge_2bodySelf[1], axis=0)
    ys = _edge_mlp(ds, edge_attr_2bodySelf, W1_s, b1_s, W2_s, b2_s, blk=2000)
    vs = jax.ops.segment_sum(ys, edge_2bodySelf[1], num_segments=n)
    return v2 + v3 + vs
